# baseline (speedup 1.0000x reference)
import functools

import numpy as np

import jax
import jax.numpy as jnp
from jax import lax
from jax.experimental import pallas as pl
from jax.experimental.pallas import tpu as pltpu
from jax.experimental.pallas import tpu_sc as plsc

N = 10000
D = 128
E = 320000
NPAD = 10240
TOTC = 160
EPAD = 16 * TOTC * 128
NPT = NPAD // 16
TRASH = NPAD - 1
CPC = TOTC // 2
_PASS = 40
FAST_CID = 1

_PERM = np.concatenate([
    np.concatenate([np.arange(32 * b, 32 * b + 32, 2),
                    np.arange(32 * b + 1, 32 * b + 32, 2)])
    for b in range(4)
])

_mesh = plsc.VectorSubcoreMesh(core_axis_name="c", subcore_axis_name="s")


def _wid_base(n_cores):
    cid = lax.axis_index("c")
    sid = lax.axis_index("s")
    return sid * n_cores + cid, cid, sid


def _sc_agg_body(srcs, dsts, xp, degp_out, aggp_out,
                 src_v, dst_v, bf0, bf1, ones_v, zrow_v,
                 deg_s, agg_s, g0, g1, sd, ss, sd2, sd3):
    wid, cid, sid = _wid_base(2)
    base = sid * NPT

    for k in range(8):
        ones_v[pl.ds(k * 16, 16)] = jnp.full((16,), 1.0, jnp.float32)

    def zfill(i, _):
        for k in range(8):
            bf0[i, pl.ds(k * 16, 16)] = jnp.zeros((16,), jnp.float32)
        return 0
    lax.fori_loop(0, 128, zfill, 0)

    def zfill1(i, _):
        zrow_v[pl.ds(i * 16, 16)] = jnp.zeros((16,), jnp.float32)
        return 0
    lax.fori_loop(0, NPT // 16, zfill1, 0)

    pltpu.sync_copy(zrow_v, deg_s.at[pl.ds(base, NPT)])

    def zago(i, _):
        pltpu.sync_copy(bf0, agg_s.at[pl.ds(base + i * 128, 128), :])
        return 0
    lax.fori_loop(0, NPT // 128, zago, 0)

    plsc.subcore_barrier()

    def _issue(j, rows, sa, sb):
        pltpu.async_copy(xp.at[src_v.at[j, pl.ds(0, 64)]],
                         rows.at[pl.ds(0, 64)], sa)
        pltpu.async_copy(xp.at[src_v.at[j, pl.ds(64, 64)]],
                         rows.at[pl.ds(64, 64)], sb)

    def _wait(j, rows, sa, sb):
        pltpu.make_async_copy(xp.at[src_v.at[j, pl.ds(0, 64)]],
                              rows.at[pl.ds(0, 64)], sa).wait()
        pltpu.make_async_copy(xp.at[src_v.at[j, pl.ds(64, 64)]],
                              rows.at[pl.ds(64, 64)], sb).wait()

    def _do_chunk(j, n, rows, sa, sb):
        pltpu.sync_copy(rows, agg_s.at[dst_v.at[j]], add=True)

        @pl.when(j + 2 < n)
        def _():
            _issue(j + 2, rows, sa, sb)

    def _load_run(off, n):
        ld0 = pltpu.async_copy(srcs.at[sid, pl.ds(off, n)],
                               src_v.at[pl.ds(0, n)], ss)
        ld1 = pltpu.async_copy(dsts.at[sid, pl.ds(off, n)],
                               dst_v.at[pl.ds(0, n)], ss)
        ld0.wait()
        ld1.wait()
        _issue(0, bf0, g0, sd2)
        _issue(1, bf1, g1, sd3)

        def pair(i, _):
            j0 = 2 * i
            _wait(j0, bf0, g0, sd2)
            _do_chunk(j0, n, bf0, g0, sd2)
            _wait(j0 + 1, bf1, g1, sd3)
            _do_chunk(j0 + 1, n, bf1, g1, sd3)
            return 0
        lax.fori_loop(0, n // 2, pair, 0)

    def _deg_run(off, n):
        ld1 = pltpu.async_copy(dsts.at[sid, pl.ds(off, n)],
                               dst_v.at[pl.ds(0, n)], ss)
        ld1.wait()

        def dchunk(j, _):
            dd = pltpu.async_copy(ones_v, deg_s.at[dst_v.at[j]], sd,
                                  add=True)
            dd.wait()
            return 0
        lax.fori_loop(0, n, dchunk, 0)

    @pl.when(cid == FAST_CID)
    def _():
        for p in range(TOTC // _PASS):
            _load_run(p * _PASS, _PASS)

    @pl.when(cid != FAST_CID)
    def _():
        for p in range(TOTC // _PASS):
            _deg_run(p * _PASS, _PASS)

    plsc.subcore_barrier()

    pltpu.sync_copy(deg_s.at[pl.ds(base, NPT)], degp_out.at[cid, pl.ds(base, NPT)])
    pltpu.sync_copy(agg_s.at[pl.ds(base, NPT), :],
                    aggp_out.at[cid, pl.ds(base, NPT), :])


_sc_agg = pl.kernel(
    _sc_agg_body,
    out_type=(
        jax.ShapeDtypeStruct((2, NPAD), jnp.float32),
        jax.ShapeDtypeStruct((2, NPAD, D), jnp.float32),
    ),
    mesh=_mesh,
    scratch_types=[
        pltpu.VMEM((_PASS, 128), jnp.int32),
        pltpu.VMEM((_PASS, 128), jnp.int32),
        pltpu.VMEM((128, D), jnp.float32),
        pltpu.VMEM((128, D), jnp.float32),
        pltpu.VMEM((128,), jnp.float32),
        pltpu.VMEM((NPT,), jnp.float32),
        pltpu.VMEM_SHARED((NPAD,), jnp.float32),
        pltpu.VMEM_SHARED((NPAD, D), jnp.float32),
        pltpu.SemaphoreType.DMA,
        pltpu.SemaphoreType.DMA,
        pltpu.SemaphoreType.DMA,
        pltpu.SemaphoreType.DMA,
        pltpu.SemaphoreType.DMA,
        pltpu.SemaphoreType.DMA,
    ],
    compiler_params=pltpu.CompilerParams(needs_layout_passes=False),
)


def _sc_coef_body(srcs, dsts, degp, inv_out, cp_out,
                  src_v, dst_v, d0_v, d1_v, inv_v, w_v, zrow_v,
                  c_s, sem):
    wid, cid, sid = _wid_base(2)
    base = sid * NPT

    def zfill1(i, _):
        zrow_v[pl.ds(i * 16, 16)] = jnp.zeros((16,), jnp.float32)
        return 0
    lax.fori_loop(0, NPT // 16, zfill1, 0)

    pltpu.sync_copy(zrow_v, c_s.at[pl.ds(base, NPT)])

    pltpu.sync_copy(degp.at[0], d0_v)
    pltpu.sync_copy(degp.at[1], d1_v)

    def invf(i, _):
        v = d0_v[pl.ds(i * 16, 16)] + d1_v[pl.ds(i * 16, 16)]
        inv_v[pl.ds(i * 16, 16)] = 1.0 / jnp.maximum(v, 1.0)
        return 0
    lax.fori_loop(0, NPAD // 16, invf, 0)

    pltpu.sync_copy(srcs.at[sid, pl.ds(cid * (TOTC // 2), TOTC // 2)], src_v)
    pltpu.sync_copy(dsts.at[sid, pl.ds(cid * (TOTC // 2), TOTC // 2)], dst_v)

    plsc.subcore_barrier()

    def chunk(j, _):
        for k in range(8):
            idx = dst_v[j, pl.ds(k * 16, 16)]
            w_v[pl.ds(k * 16, 16)] = plsc.load_gather(inv_v, [idx])
        pltpu.sync_copy(w_v, c_s.at[src_v.at[j]], add=True)
        return 0
    lax.fori_loop(0, TOTC // 2, chunk, 0)

    plsc.subcore_barrier()

    pltpu.sync_copy(c_s.at[pl.ds(base, NPT)], cp_out.at[cid, pl.ds(base, NPT)])

    @pl.when(cid == 0)
    def _():
        pltpu.sync_copy(inv_v.at[pl.ds(base, NPT)], inv_out.at[pl.ds(base, NPT)])


_sc_coef = pl.kernel(
    _sc_coef_body,
    out_type=(
        jax.ShapeDtypeStruct((NPAD,), jnp.float32),
        jax.ShapeDtypeStruct((2, NPAD), jnp.float32),
    ),
    mesh=_mesh,
    scratch_types=[
        pltpu.VMEM((TOTC // 2, 128), jnp.int32),
        pltpu.VMEM((TOTC // 2, 128), jnp.int32),
        pltpu.VMEM((NPAD,), jnp.float32),
        pltpu.VMEM((NPAD,), jnp.float32),
        pltpu.VMEM((NPAD,), jnp.float32),
        pltpu.VMEM((128,), jnp.float32),
        pltpu.VMEM((NPT,), jnp.float32),
        pltpu.VMEM_SHARED((NPAD,), jnp.float32),
        pltpu.SemaphoreType.DMA,
    ],
    compiler_params=pltpu.CompilerParams(needs_layout_passes=False),
)


_BLK = 128
_NBLK = NPAD // _BLK


def _dotT(a, b):
    return lax.dot_general(a, b, (((1,), (1,)), ((), ())),
                           preferred_element_type=jnp.float32,
                           precision=lax.Precision.HIGHEST)


def _tc_body(xp_ref, aggp_ref, invr_ref, cpr_ref,
             w1l_ref, b1_ref, w1r_ref, w2lp_ref, b2p_ref, w2rp_ref,
             out_ref, s2_acc, sh_acc):
    i = pl.program_id(0)

    @pl.when(i == 0)
    def _():
        s2_acc[...] = jnp.zeros((1, 128), jnp.float32)
        sh_acc[...] = jnp.zeros((1, 128), jnp.float32)

    agg = aggp_ref[0] + aggp_ref[1]
    inv_row = invr_ref[0]
    c_row = cpr_ref[0, 0] + cpr_ref[1, 0]

    rows = lax.broadcasted_iota(jnp.int32, (_BLK, _BLK), 0)
    cols = lax.broadcasted_iota(jnp.int32, (_BLK, _BLK), 1)
    eye = jnp.where(rows == cols, 1.0, 0.0).astype(jnp.float32)
    inv_col = _dotT(eye, inv_row)
    c_col = _dotT(eye, c_row)

    rowid = i * _BLK + lax.broadcasted_iota(jnp.int32, (_BLK, 1), 0)
    mask = rowid < N

    h = _dotT(agg * inv_col, w1l_ref[0]) + _dotT(xp_ref[...], w1r_ref[0])
    h = jax.nn.relu(h + b1_ref[0])
    h = jnp.where(mask, h, 0.0)

    sh_acc[...] += jnp.sum(h, axis=0, keepdims=True)
    s2_acc[...] += jnp.sum(h * c_col, axis=0, keepdims=True)

    @pl.when(i == _NBLK - 1)
    def _():
        o = (_dotT(s2_acc[...], w2lp_ref[0]) + float(N) * b2p_ref[0]
             + _dotT(sh_acc[...], w2rp_ref[0]))
        out_ref[...] = jnp.broadcast_to(o, (8, 128))


_tc_dense = pl.pallas_call(
    _tc_body,
    grid=(_NBLK,),
    in_specs=[
        pl.BlockSpec((_BLK, D), lambda i: (i, 0)),
        pl.BlockSpec((2, _BLK, D), lambda i: (0, i, 0)),
        pl.BlockSpec((1, 1, 128), lambda i: (i, 0, 0)),
        pl.BlockSpec((2, 1, 1, 128), lambda i: (0, i, 0, 0)),
        pl.BlockSpec((1, D, D), lambda i: (0, 0, 0)),
        pl.BlockSpec((1, 1, 128), lambda i: (0, 0, 0)),
        pl.BlockSpec((1, D, D), lambda i: (0, 0, 0)),
        pl.BlockSpec((1, D, D), lambda i: (0, 0, 0)),
        pl.BlockSpec((1, 1, 128), lambda i: (0, 0, 0)),
        pl.BlockSpec((1, D, D), lambda i: (0, 0, 0)),
    ],
    out_specs=pl.BlockSpec((8, 128), lambda i: (0, 0)),
    out_shape=jax.ShapeDtypeStruct((8, 128), jnp.float32),
    scratch_shapes=[
        pltpu.VMEM((1, 128), jnp.float32),
        pltpu.VMEM((1, 128), jnp.float32),
    ],
)


def kernel(x, edge_index, W1_l, b1, W1_r, W2_l, b2, W2_r):
    src = edge_index[0].astype(jnp.int32)
    dst = edge_index[1].astype(jnp.int32)
    fill = jnp.full((EPAD - E,), TRASH, jnp.int32)
    srcs = jnp.concatenate([src, fill]).reshape(16, TOTC, 128)
    dsts = jnp.concatenate([dst, fill]).reshape(16, TOTC, 128)
    xp = jnp.pad(x, ((0, NPAD - N), (0, 0)))

    degp, aggp = _sc_agg(srcs, dsts, xp)
    inv, cp = _sc_coef(srcs, dsts, degp)

    w1lp = W1_l
    w2lp = jnp.pad(W2_l, ((0, D - W2_l.shape[0]), (0, 0)))
    w2rp = jnp.pad(W2_r, ((0, D - W2_r.shape[0]), (0, 0)))
    b2p = jnp.pad(b2, (0, 128 - b2.shape[0])).reshape(1, 1, 128)

    out = _tc_dense(
        xp, aggp,
        inv.reshape(_NBLK, 1, 128), cp.reshape(2, _NBLK, 1, 128),
        w1lp.reshape(1, D, D), b1.reshape(1, 1, 128), W1_r.reshape(1, D, D),
        w2lp.reshape(1, D, D), b2p, w2rp.reshape(1, D, D),
    )
    return out[0:1, 0:10]


# speedup vs baseline: 1.2070x; 1.2070x over previous
import functools

import jax
import jax.numpy as jnp
from jax import lax
from jax.experimental import pallas as pl
from jax.experimental.pallas import tpu as pltpu
from jax.experimental.pallas import tpu_sc as plsc

N = 10000
D = 128
E = 320000
NPAD = 10240
TOTC = 160
EPAD = 16 * TOTC * 128
NPT = NPAD // 16
TRASH = NPAD - 1
CPC = TOTC // 2
_PASS = 40
FAST_CID = 0
CHF = 120
CHS = TOTC - CHF

_mesh = plsc.VectorSubcoreMesh(core_axis_name="c", subcore_axis_name="s")


def _wid_base(n_cores):
    cid = lax.axis_index("c")
    sid = lax.axis_index("s")
    return sid * n_cores + cid, cid, sid


def _sc_agg_body(srcs, dsts, xp, degp_out, aggp_out,
                 src_v, dst_v, bf0, bf1, ones_v, zrow_v,
                 deg_s, agg_s, g0, g1, sd, ss):
    wid, cid, sid = _wid_base(2)
    base = sid * NPT

    for k in range(8):
        ones_v[pl.ds(k * 16, 16)] = jnp.full((16,), 1.0, jnp.float32)

    def zfill(i, _):
        for k in range(8):
            bf0[i, pl.ds(k * 16, 16)] = jnp.zeros((16,), jnp.float32)
        return 0
    lax.fori_loop(0, 128, zfill, 0)

    def zfill1(i, _):
        zrow_v[pl.ds(i * 16, 16)] = jnp.zeros((16,), jnp.float32)
        return 0
    lax.fori_loop(0, NPT // 16, zfill1, 0)

    pltpu.sync_copy(zrow_v, deg_s.at[pl.ds(base, NPT)])

    def zago(i, _):
        pltpu.sync_copy(bf0, agg_s.at[pl.ds(base + i * 128, 128), :])
        return 0
    lax.fori_loop(0, NPT // 128, zago, 0)

    plsc.subcore_barrier()

    def _do_chunk(j, n, rows, gsem):
        dd = pltpu.async_copy(ones_v, deg_s.at[dst_v.at[j]], sd, add=True)
        pltpu.sync_copy(rows, agg_s.at[dst_v.at[j]], add=True)
        dd.wait()

        @pl.when(j + 2 < n)
        def _():
            pltpu.async_copy(xp.at[src_v.at[j + 2]], rows, gsem)

    def _load_run(off, n):
        ld0 = pltpu.async_copy(srcs.at[sid, pl.ds(off, n)],
                               src_v.at[pl.ds(0, n)], ss)
        ld1 = pltpu.async_copy(dsts.at[sid, pl.ds(off, n)],
                               dst_v.at[pl.ds(0, n)], ss)
        ld0.wait()
        ld1.wait()
        pltpu.async_copy(xp.at[src_v.at[0]], bf0, g0)
        pltpu.async_copy(xp.at[src_v.at[1]], bf1, g1)

        def pair(i, _):
            j0 = 2 * i
            pltpu.make_async_copy(xp.at[src_v.at[j0]], bf0, g0).wait()
            _do_chunk(j0, n, bf0, g0)
            pltpu.make_async_copy(xp.at[src_v.at[j0 + 1]], bf1, g1).wait()
            _do_chunk(j0 + 1, n, bf1, g1)
            return 0
        lax.fori_loop(0, n // 2, pair, 0)

    @pl.when(cid == FAST_CID)
    def _():
        for p in range(CHF // _PASS):
            _load_run(p * _PASS, _PASS)

    @pl.when(cid != FAST_CID)
    def _():
        _load_run(CHF, CHS)

    plsc.subcore_barrier()

    pltpu.sync_copy(deg_s.at[pl.ds(base, NPT)], degp_out.at[cid, pl.ds(base, NPT)])
    pltpu.sync_copy(agg_s.at[pl.ds(base, NPT), :],
                    aggp_out.at[cid, pl.ds(base, NPT), :])


_sc_agg = pl.kernel(
    _sc_agg_body,
    out_type=(
        jax.ShapeDtypeStruct((2, NPAD), jnp.float32),
        jax.ShapeDtypeStruct((2, NPAD, D), jnp.float32),
    ),
    mesh=_mesh,
    scratch_types=[
        pltpu.VMEM((_PASS, 128), jnp.int32),
        pltpu.VMEM((_PASS, 128), jnp.int32),
        pltpu.VMEM((128, D), jnp.float32),
        pltpu.VMEM((128, D), jnp.float32),
        pltpu.VMEM((128,), jnp.float32),
        pltpu.VMEM((NPT,), jnp.float32),
        pltpu.VMEM_SHARED((NPAD,), jnp.float32),
        pltpu.VMEM_SHARED((NPAD, D), jnp.float32),
        pltpu.SemaphoreType.DMA,
        pltpu.SemaphoreType.DMA,
        pltpu.SemaphoreType.DMA,
        pltpu.SemaphoreType.DMA,
    ],
    compiler_params=pltpu.CompilerParams(needs_layout_passes=False),
)


def _sc_coef_body(srcs, dsts, degp, inv_out, cp_out,
                  src_v, dst_v, d0_v, d1_v, inv_v, w_v, zrow_v,
                  c_s, sem):
    wid, cid, sid = _wid_base(2)
    base = sid * NPT

    def zfill1(i, _):
        zrow_v[pl.ds(i * 16, 16)] = jnp.zeros((16,), jnp.float32)
        return 0
    lax.fori_loop(0, NPT // 16, zfill1, 0)

    pltpu.sync_copy(zrow_v, c_s.at[pl.ds(base, NPT)])

    pltpu.sync_copy(degp.at[0], d0_v)
    pltpu.sync_copy(degp.at[1], d1_v)

    def invf(i, _):
        v = d0_v[pl.ds(i * 16, 16)] + d1_v[pl.ds(i * 16, 16)]
        inv_v[pl.ds(i * 16, 16)] = 1.0 / jnp.maximum(v, 1.0)
        return 0
    lax.fori_loop(0, NPAD // 16, invf, 0)

    pltpu.sync_copy(srcs.at[sid, pl.ds(cid * (TOTC // 2), TOTC // 2)], src_v)
    pltpu.sync_copy(dsts.at[sid, pl.ds(cid * (TOTC // 2), TOTC // 2)], dst_v)

    plsc.subcore_barrier()

    def chunk(j, _):
        for k in range(8):
            idx = dst_v[j, pl.ds(k * 16, 16)]
            w_v[pl.ds(k * 16, 16)] = plsc.load_gather(inv_v, [idx])
        pltpu.sync_copy(w_v, c_s.at[src_v.at[j]], add=True)
        return 0
    lax.fori_loop(0, TOTC // 2, chunk, 0)

    plsc.subcore_barrier()

    pltpu.sync_copy(c_s.at[pl.ds(base, NPT)], cp_out.at[cid, pl.ds(base, NPT)])

    @pl.when(cid == 0)
    def _():
        pltpu.sync_copy(inv_v.at[pl.ds(base, NPT)], inv_out.at[pl.ds(base, NPT)])


_sc_coef = pl.kernel(
    _sc_coef_body,
    out_type=(
        jax.ShapeDtypeStruct((NPAD,), jnp.float32),
        jax.ShapeDtypeStruct((2, NPAD), jnp.float32),
    ),
    mesh=_mesh,
    scratch_types=[
        pltpu.VMEM((TOTC // 2, 128), jnp.int32),
        pltpu.VMEM((TOTC // 2, 128), jnp.int32),
        pltpu.VMEM((NPAD,), jnp.float32),
        pltpu.VMEM((NPAD,), jnp.float32),
        pltpu.VMEM((NPAD,), jnp.float32),
        pltpu.VMEM((128,), jnp.float32),
        pltpu.VMEM((NPT,), jnp.float32),
        pltpu.VMEM_SHARED((NPAD,), jnp.float32),
        pltpu.SemaphoreType.DMA,
    ],
    compiler_params=pltpu.CompilerParams(needs_layout_passes=False),
)


_BLK = 128
_NBLK = NPAD // _BLK


def _dotT(a, b):
    return lax.dot_general(a, b, (((1,), (1,)), ((), ())),
                           preferred_element_type=jnp.float32,
                           precision=lax.Precision.HIGHEST)


def _tc_body(xp_ref, aggp_ref, invr_ref, cpr_ref,
             w1l_ref, b1_ref, w1r_ref, w2lp_ref, b2p_ref, w2rp_ref,
             out_ref, s2_acc, sh_acc):
    i = pl.program_id(0)

    @pl.when(i == 0)
    def _():
        s2_acc[...] = jnp.zeros((1, 128), jnp.float32)
        sh_acc[...] = jnp.zeros((1, 128), jnp.float32)

    agg = aggp_ref[0] + aggp_ref[1]
    inv_row = invr_ref[0]
    c_row = cpr_ref[0, 0] + cpr_ref[1, 0]

    rows = lax.broadcasted_iota(jnp.int32, (_BLK, _BLK), 0)
    cols = lax.broadcasted_iota(jnp.int32, (_BLK, _BLK), 1)
    eye = jnp.where(rows == cols, 1.0, 0.0).astype(jnp.float32)
    inv_col = _dotT(eye, inv_row)
    c_col = _dotT(eye, c_row)

    rowid = i * _BLK + lax.broadcasted_iota(jnp.int32, (_BLK, 1), 0)
    mask = rowid < N

    h = _dotT(agg * inv_col, w1l_ref[0]) + _dotT(xp_ref[...], w1r_ref[0])
    h = jax.nn.relu(h + b1_ref[0])
    h = jnp.where(mask, h, 0.0)

    sh_acc[...] += jnp.sum(h, axis=0, keepdims=True)
    s2_acc[...] += jnp.sum(h * c_col, axis=0, keepdims=True)

    @pl.when(i == _NBLK - 1)
    def _():
        o = (_dotT(s2_acc[...], w2lp_ref[0]) + float(N) * b2p_ref[0]
             + _dotT(sh_acc[...], w2rp_ref[0]))
        out_ref[...] = jnp.broadcast_to(o, (8, 128))


_tc_dense = pl.pallas_call(
    _tc_body,
    grid=(_NBLK,),
    in_specs=[
        pl.BlockSpec((_BLK, D), lambda i: (i, 0)),
        pl.BlockSpec((2, _BLK, D), lambda i: (0, i, 0)),
        pl.BlockSpec((1, 1, 128), lambda i: (i, 0, 0)),
        pl.BlockSpec((2, 1, 1, 128), lambda i: (0, i, 0, 0)),
        pl.BlockSpec((1, D, D), lambda i: (0, 0, 0)),
        pl.BlockSpec((1, 1, 128), lambda i: (0, 0, 0)),
        pl.BlockSpec((1, D, D), lambda i: (0, 0, 0)),
        pl.BlockSpec((1, D, D), lambda i: (0, 0, 0)),
        pl.BlockSpec((1, 1, 128), lambda i: (0, 0, 0)),
        pl.BlockSpec((1, D, D), lambda i: (0, 0, 0)),
    ],
    out_specs=pl.BlockSpec((8, 128), lambda i: (0, 0)),
    out_shape=jax.ShapeDtypeStruct((8, 128), jnp.float32),
    scratch_shapes=[
        pltpu.VMEM((1, 128), jnp.float32),
        pltpu.VMEM((1, 128), jnp.float32),
    ],
)


def kernel(x, edge_index, W1_l, b1, W1_r, W2_l, b2, W2_r):
    src = edge_index[0].astype(jnp.int32)
    dst = edge_index[1].astype(jnp.int32)
    fill = jnp.full((EPAD - E,), TRASH, jnp.int32)
    srcs = jnp.concatenate([src, fill]).reshape(16, TOTC, 128)
    dsts = jnp.concatenate([dst, fill]).reshape(16, TOTC, 128)
    xp = jnp.pad(x, ((0, NPAD - N), (0, 0)))

    degp, aggp = _sc_agg(srcs, dsts, xp)
    inv, cp = _sc_coef(srcs, dsts, degp)

    w1lp = W1_l
    w2lp = jnp.pad(W2_l, ((0, D - W2_l.shape[0]), (0, 0)))
    w2rp = jnp.pad(W2_r, ((0, D - W2_r.shape[0]), (0, 0)))
    b2p = jnp.pad(b2, (0, 128 - b2.shape[0])).reshape(1, 1, 128)

    out = _tc_dense(
        xp, aggp,
        inv.reshape(_NBLK, 1, 128), cp.reshape(2, _NBLK, 1, 128),
        w1lp.reshape(1, D, D), b1.reshape(1, 1, 128), W1_r.reshape(1, D, D),
        w2lp.reshape(1, D, D), b2p, w2rp.reshape(1, D, D),
    )
    return out[0:1, 0:10]


# speedup vs baseline: 1.2256x; 1.0154x over previous
import functools

import jax
import jax.numpy as jnp
from jax import lax
from jax.experimental import pallas as pl
from jax.experimental.pallas import tpu as pltpu
from jax.experimental.pallas import tpu_sc as plsc

N = 10000
D = 128
E = 320000
NPAD = 10240
TOTC = 160
EPAD = 16 * TOTC * 128
NPT = NPAD // 16
TRASH = NPAD - 1
CPC = TOTC // 2
_PASS = 40
FAST_CID = 0
CHF = 120
CHS = TOTC - CHF

_mesh = plsc.VectorSubcoreMesh(core_axis_name="c", subcore_axis_name="s")


def _wid_base(n_cores):
    cid = lax.axis_index("c")
    sid = lax.axis_index("s")
    return sid * n_cores + cid, cid, sid


def _sc_agg_body(srcs, dsts, xp, degp_out, aggp_out,
                 src_v, dst_v, bf0, bf1, ones_v, zrow_v,
                 deg_s, agg_s, g0, g1, sd, ss):
    wid, cid, sid = _wid_base(2)
    base = sid * NPT

    for k in range(8):
        ones_v[pl.ds(k * 16, 16)] = jnp.full((16,), 1.0, jnp.float32)

    def zfill(i, _):
        for k in range(8):
            bf0[i, pl.ds(k * 16, 16)] = jnp.zeros((16,), jnp.float32)
        return 0
    lax.fori_loop(0, 128, zfill, 0)

    def zfill1(i, _):
        zrow_v[pl.ds(i * 16, 16)] = jnp.zeros((16,), jnp.float32)
        return 0
    lax.fori_loop(0, NPT // 16, zfill1, 0)

    pltpu.sync_copy(zrow_v, deg_s.at[pl.ds(base, NPT)])

    def zago(i, _):
        pltpu.sync_copy(bf0, agg_s.at[pl.ds(base + i * 128, 128), :])
        return 0
    lax.fori_loop(0, NPT // 128, zago, 0)

    plsc.subcore_barrier()

    def _do_chunk(j, n, rows, gsem):
        dd = pltpu.async_copy(ones_v, deg_s.at[dst_v.at[j]], sd, add=True)
        pltpu.sync_copy(rows, agg_s.at[dst_v.at[j]], add=True)
        dd.wait()

        @pl.when(j + 2 < n)
        def _():
            pltpu.async_copy(xp.at[src_v.at[j + 2]], rows, gsem)

    def _load_run(off, n):
        ld0 = pltpu.async_copy(srcs.at[sid, pl.ds(off, n)],
                               src_v.at[pl.ds(0, n)], ss)
        ld1 = pltpu.async_copy(dsts.at[sid, pl.ds(off, n)],
                               dst_v.at[pl.ds(0, n)], ss)
        ld0.wait()
        ld1.wait()
        pltpu.async_copy(xp.at[src_v.at[0]], bf0, g0)
        pltpu.async_copy(xp.at[src_v.at[1]], bf1, g1)

        def pair(i, _):
            j0 = 2 * i
            pltpu.make_async_copy(xp.at[src_v.at[j0]], bf0, g0).wait()
            _do_chunk(j0, n, bf0, g0)
            pltpu.make_async_copy(xp.at[src_v.at[j0 + 1]], bf1, g1).wait()
            _do_chunk(j0 + 1, n, bf1, g1)
            return 0
        lax.fori_loop(0, n // 2, pair, 0)

    @pl.when(cid == FAST_CID)
    def _():
        for p in range(CHF // _PASS):
            _load_run(p * _PASS, _PASS)

    @pl.when(cid != FAST_CID)
    def _():
        _load_run(CHF, CHS)

    plsc.subcore_barrier()

    pltpu.sync_copy(deg_s.at[pl.ds(base, NPT)], degp_out.at[cid, pl.ds(base, NPT)])
    pltpu.sync_copy(agg_s.at[pl.ds(base, NPT), :],
                    aggp_out.at[cid, pl.ds(base, NPT), :])


_sc_agg = pl.kernel(
    _sc_agg_body,
    out_type=(
        jax.ShapeDtypeStruct((2, NPAD), jnp.float32),
        jax.ShapeDtypeStruct((2, NPAD, D), jnp.float32),
    ),
    mesh=_mesh,
    scratch_types=[
        pltpu.VMEM((_PASS, 128), jnp.int32),
        pltpu.VMEM((_PASS, 128), jnp.int32),
        pltpu.VMEM((128, D), jnp.float32),
        pltpu.VMEM((128, D), jnp.float32),
        pltpu.VMEM((128,), jnp.float32),
        pltpu.VMEM((NPT,), jnp.float32),
        pltpu.VMEM_SHARED((NPAD,), jnp.float32),
        pltpu.VMEM_SHARED((NPAD, D), jnp.float32),
        pltpu.SemaphoreType.DMA,
        pltpu.SemaphoreType.DMA,
        pltpu.SemaphoreType.DMA,
        pltpu.SemaphoreType.DMA,
    ],
    compiler_params=pltpu.CompilerParams(needs_layout_passes=False),
)


def _sc_coef_body(srcs, dsts, degp, inv_out, cp_out,
                  src_v, dst_v, d0_v, d1_v, inv_v, w_v, zrow_v,
                  c_s, sem):
    wid, cid, sid = _wid_base(2)
    base = sid * NPT

    def zfill1(i, _):
        zrow_v[pl.ds(i * 16, 16)] = jnp.zeros((16,), jnp.float32)
        return 0
    lax.fori_loop(0, NPT // 16, zfill1, 0)

    pltpu.sync_copy(zrow_v, c_s.at[pl.ds(base, NPT)])

    pltpu.sync_copy(degp.at[0], d0_v)
    pltpu.sync_copy(degp.at[1], d1_v)

    def invf(i, _):
        v = d0_v[pl.ds(i * 16, 16)] + d1_v[pl.ds(i * 16, 16)]
        inv_v[pl.ds(i * 16, 16)] = 1.0 / jnp.maximum(v, 1.0)
        return 0
    lax.fori_loop(0, NPAD // 16, invf, 0)

    pltpu.sync_copy(srcs.at[sid, pl.ds(cid * (TOTC // 2), TOTC // 2)], src_v)
    pltpu.sync_copy(dsts.at[sid, pl.ds(cid * (TOTC // 2), TOTC // 2)], dst_v)

    plsc.subcore_barrier()

    def chunk(j, _):
        for k in range(8):
            idx = dst_v[j, pl.ds(k * 16, 16)]
            w_v[pl.ds(k * 16, 16)] = plsc.load_gather(inv_v, [idx])
        pltpu.sync_copy(w_v, c_s.at[src_v.at[j]], add=True)
        return 0
    lax.fori_loop(0, TOTC // 2, chunk, 0)

    plsc.subcore_barrier()

    pltpu.sync_copy(c_s.at[pl.ds(base, NPT)], cp_out.at[cid, pl.ds(base, NPT)])

    @pl.when(cid == 0)
    def _():
        pltpu.sync_copy(inv_v.at[pl.ds(base, NPT)], inv_out.at[pl.ds(base, NPT)])


_sc_coef = pl.kernel(
    _sc_coef_body,
    out_type=(
        jax.ShapeDtypeStruct((NPAD,), jnp.float32),
        jax.ShapeDtypeStruct((2, NPAD), jnp.float32),
    ),
    mesh=_mesh,
    scratch_types=[
        pltpu.VMEM((TOTC // 2, 128), jnp.int32),
        pltpu.VMEM((TOTC // 2, 128), jnp.int32),
        pltpu.VMEM((NPAD,), jnp.float32),
        pltpu.VMEM((NPAD,), jnp.float32),
        pltpu.VMEM((NPAD,), jnp.float32),
        pltpu.VMEM((128,), jnp.float32),
        pltpu.VMEM((NPT,), jnp.float32),
        pltpu.VMEM_SHARED((NPAD,), jnp.float32),
        pltpu.SemaphoreType.DMA,
    ],
    compiler_params=pltpu.CompilerParams(needs_layout_passes=False),
)


_BLK = 128
_NBLK = NPAD // _BLK


def _dotT(a, b):
    return lax.dot_general(a, b, (((1,), (1,)), ((), ())),
                           preferred_element_type=jnp.float32,
                           precision=lax.Precision.DEFAULT)


def _tc_body(xp_ref, aggp_ref, invr_ref, cpr_ref,
             w1l_ref, b1_ref, w1r_ref, w2lp_ref, b2p_ref, w2rp_ref,
             out_ref, s2_acc, sh_acc):
    i = pl.program_id(0)

    @pl.when(i == 0)
    def _():
        s2_acc[...] = jnp.zeros((1, 128), jnp.float32)
        sh_acc[...] = jnp.zeros((1, 128), jnp.float32)

    agg = aggp_ref[0] + aggp_ref[1]
    inv_row = invr_ref[0]
    c_row = cpr_ref[0, 0] + cpr_ref[1, 0]

    rows = lax.broadcasted_iota(jnp.int32, (_BLK, _BLK), 0)
    cols = lax.broadcasted_iota(jnp.int32, (_BLK, _BLK), 1)
    eye = jnp.where(rows == cols, 1.0, 0.0).astype(jnp.float32)
    inv_col = _dotT(eye, inv_row)
    c_col = _dotT(eye, c_row)

    rowid = i * _BLK + lax.broadcasted_iota(jnp.int32, (_BLK, 1), 0)
    mask = rowid < N

    h = _dotT(agg * inv_col, w1l_ref[0]) + _dotT(xp_ref[...], w1r_ref[0])
    h = jax.nn.relu(h + b1_ref[0])
    h = jnp.where(mask, h, 0.0)

    sh_acc[...] += jnp.sum(h, axis=0, keepdims=True)
    s2_acc[...] += jnp.sum(h * c_col, axis=0, keepdims=True)

    @pl.when(i == _NBLK - 1)
    def _():
        o = (_dotT(s2_acc[...], w2lp_ref[0]) + float(N) * b2p_ref[0]
             + _dotT(sh_acc[...], w2rp_ref[0]))
        out_ref[...] = jnp.broadcast_to(o, (8, 128))


_tc_dense = pl.pallas_call(
    _tc_body,
    grid=(_NBLK,),
    in_specs=[
        pl.BlockSpec((_BLK, D), lambda i: (i, 0)),
        pl.BlockSpec((2, _BLK, D), lambda i: (0, i, 0)),
        pl.BlockSpec((1, 1, 128), lambda i: (i, 0, 0)),
        pl.BlockSpec((2, 1, 1, 128), lambda i: (0, i, 0, 0)),
        pl.BlockSpec((1, D, D), lambda i: (0, 0, 0)),
        pl.BlockSpec((1, 1, 128), lambda i: (0, 0, 0)),
        pl.BlockSpec((1, D, D), lambda i: (0, 0, 0)),
        pl.BlockSpec((1, D, D), lambda i: (0, 0, 0)),
        pl.BlockSpec((1, 1, 128), lambda i: (0, 0, 0)),
        pl.BlockSpec((1, D, D), lambda i: (0, 0, 0)),
    ],
    out_specs=pl.BlockSpec((8, 128), lambda i: (0, 0)),
    out_shape=jax.ShapeDtypeStruct((8, 128), jnp.float32),
    scratch_shapes=[
        pltpu.VMEM((1, 128), jnp.float32),
        pltpu.VMEM((1, 128), jnp.float32),
    ],
)


def kernel(x, edge_index, W1_l, b1, W1_r, W2_l, b2, W2_r):
    src = edge_index[0].astype(jnp.int32)
    dst = edge_index[1].astype(jnp.int32)
    fill = jnp.full((EPAD - E,), TRASH, jnp.int32)
    srcs = jnp.concatenate([src, fill]).reshape(16, TOTC, 128)
    dsts = jnp.concatenate([dst, fill]).reshape(16, TOTC, 128)
    xp = jnp.pad(x, ((0, NPAD - N), (0, 0)))

    degp, aggp = _sc_agg(srcs, dsts, xp)
    inv, cp = _sc_coef(srcs, dsts, degp)

    w1lp = W1_l
    w2lp = jnp.pad(W2_l, ((0, D - W2_l.shape[0]), (0, 0)))
    w2rp = jnp.pad(W2_r, ((0, D - W2_r.shape[0]), (0, 0)))
    b2p = jnp.pad(b2, (0, 128 - b2.shape[0])).reshape(1, 1, 128)

    out = _tc_dense(
        xp, aggp,
        inv.reshape(_NBLK, 1, 128), cp.reshape(2, _NBLK, 1, 128),
        w1lp.reshape(1, D, D), b1.reshape(1, 1, 128), W1_r.reshape(1, D, D),
        w2lp.reshape(1, D, D), b2p, w2rp.reshape(1, D, D),
    )
    return out[0:1, 0:10]
